# baseline (speedup 1.0000x reference)
import functools

import jax
import jax.numpy as jnp
from jax import lax
from jax.experimental import pallas as pl
from jax.experimental.pallas import tpu as pltpu
from jax.experimental.pallas import tpu_sc as plsc

BATCH = 4096
MAXLEN = 200
EMBED = 32

NC = 2
NS = 16
NW = NC * NS

ROWS_PER_W = BATCH // NW
CHUNK_ROWS = 4
CHUNK = CHUNK_ROWS * MAXLEN
NCHUNK = ROWS_PER_W // CHUNK_ROWS
GATHER_W = 100
NGATHER = CHUNK // GATHER_W


def _embed_kernel(x_hbm, tok_hbm, pos_hbm, out_hbm, idx_v, rows_v, sem):
    wid = lax.axis_index("c") * NS + lax.axis_index("s")

    def chunk_body(g, carry):
        c = wid * NCHUNK + g
        pltpu.sync_copy(x_hbm.at[c], idx_v)

        fills = [
            pltpu.async_copy(
                pos_hbm, rows_v.at[pl.ds(r * MAXLEN, MAXLEN)], sem
            )
            for r in range(CHUNK_ROWS)
        ]
        for cp in fills:
            cp.wait()

        copies = []
        for j in range(NGATHER):
            copies.append(
                pltpu.async_copy(
                    tok_hbm.at[idx_v.at[j]],
                    rows_v.at[pl.ds(j * GATHER_W, GATHER_W)],
                    sem,
                    add=True,
                )
            )
        for cp in copies:
            cp.wait()

        pltpu.sync_copy(rows_v, out_hbm.at[pl.ds(c * CHUNK, CHUNK)])
        return carry

    lax.fori_loop(0, NCHUNK, chunk_body, 0)


def kernel(x, token_table, pos_table):
    x3 = x.astype(jnp.int32).reshape(NW * NCHUNK, NGATHER, GATHER_W)
    mesh = plsc.VectorSubcoreMesh(core_axis_name="c", subcore_axis_name="s")
    run = functools.partial(
        pl.kernel,
        mesh=mesh,
        compiler_params=pltpu.CompilerParams(use_tc_tiling_on_sc=False),
        out_type=jax.ShapeDtypeStruct((BATCH * MAXLEN, EMBED), jnp.float32),
        scratch_types=[
            pltpu.VMEM((NGATHER, GATHER_W), jnp.int32),
            pltpu.VMEM((CHUNK, EMBED), jnp.float32),
            pltpu.SemaphoreType.DMA,
        ],
    )(_embed_kernel)
    out = run(x3, token_table, pos_table)
    return out.reshape(BATCH, MAXLEN, EMBED)


# speedup vs baseline: 1.6858x; 1.6858x over previous
import functools

import jax
import jax.numpy as jnp
from jax import lax
from jax.experimental import pallas as pl
from jax.experimental.pallas import tpu as pltpu
from jax.experimental.pallas import tpu_sc as plsc

BATCH = 4096
MAXLEN = 200
EMBED = 32

NC = 2
NS = 16
NW = NC * NS

ROWS_PER_W = BATCH // NW
CHUNK_ROWS = 4
CHUNK = CHUNK_ROWS * MAXLEN
NCHUNK = ROWS_PER_W // CHUNK_ROWS
GATHER_W = 100
NGATHER = CHUNK // GATHER_W


def _embed_kernel(x_hbm, tok_hbm, pos_hbm, out_hbm,
                  idx_v, rows_v, pos_v, gsem0, gsem1, osem0, osem1):
    wid = lax.axis_index("c") * NS + lax.axis_index("s")
    gsems = (gsem0, gsem1)
    osems = (osem0, osem1)

    pltpu.sync_copy(pos_hbm, pos_v)

    def start_gathers(c, b):
        for j in range(NGATHER):
            pltpu.async_copy(
                tok_hbm.at[idx_v.at[b, j]],
                rows_v.at[b, pl.ds(j * GATHER_W, GATHER_W)],
                gsems[b],
            )

    def wait_gathers(b):
        for j in range(NGATHER):
            pltpu.make_async_copy(
                tok_hbm.at[idx_v.at[b, j]],
                rows_v.at[b, pl.ds(j * GATHER_W, GATHER_W)],
                gsems[b],
            ).wait()

    def wait_writeout(c, b):
        pltpu.make_async_copy(
            rows_v.at[b], out_hbm.at[pl.ds(c * CHUNK, CHUNK)], osems[b]
        ).wait()

    def add_pos(b):
        def add_body(t, carry2):
            p0 = pos_v[t, pl.ds(0, 16)]
            p1 = pos_v[t, pl.ds(16, 16)]
            for r in range(CHUNK_ROWS):
                row = r * MAXLEN + t
                rows_v[b, row, pl.ds(0, 16)] = rows_v[b, row, pl.ds(0, 16)] + p0
                rows_v[b, row, pl.ds(16, 16)] = rows_v[b, row, pl.ds(16, 16)] + p1
            return carry2

        lax.fori_loop(0, MAXLEN, add_body, 0)

    c0 = wid * NCHUNK
    pltpu.sync_copy(x_hbm.at[c0], idx_v.at[0])
    start_gathers(c0, 0)

    def pair_body(h, carry):
        for b in range(2):
            g = 2 * h + b
            c = c0 + g
            q = 1 - b
            wait_gathers(b)

            @pl.when(g < NCHUNK - 1)
            def _():
                pltpu.sync_copy(x_hbm.at[c + 1], idx_v.at[q])

                @pl.when(g >= 1)
                def _():
                    wait_writeout(c - 1, q)

                start_gathers(c + 1, q)

            add_pos(b)
            pltpu.async_copy(
                rows_v.at[b], out_hbm.at[pl.ds(c * CHUNK, CHUNK)], osems[b]
            )
        return carry

    lax.fori_loop(0, NCHUNK // 2, pair_body, 0)

    wait_writeout(c0 + NCHUNK - 1, 1)


def kernel(x, token_table, pos_table):
    x3 = x.astype(jnp.int32).reshape(NW * NCHUNK, NGATHER, GATHER_W)
    mesh = plsc.VectorSubcoreMesh(core_axis_name="c", subcore_axis_name="s")
    run = functools.partial(
        pl.kernel,
        mesh=mesh,
        compiler_params=pltpu.CompilerParams(use_tc_tiling_on_sc=False),
        out_type=jax.ShapeDtypeStruct((BATCH * MAXLEN, EMBED), jnp.float32),
        scratch_types=[
            pltpu.VMEM((2, NGATHER, GATHER_W), jnp.int32),
            pltpu.VMEM((2, CHUNK, EMBED), jnp.float32),
            pltpu.VMEM((MAXLEN, EMBED), jnp.float32),
            pltpu.SemaphoreType.DMA,
            pltpu.SemaphoreType.DMA,
            pltpu.SemaphoreType.DMA,
            pltpu.SemaphoreType.DMA,
        ],
    )(_embed_kernel)
    out = run(x3, token_table, pos_table)
    return out.reshape(BATCH, MAXLEN, EMBED)


# speedup vs baseline: 1.6869x; 1.0007x over previous
import functools

import jax
import jax.numpy as jnp
from jax import lax
from jax.experimental import pallas as pl
from jax.experimental.pallas import tpu as pltpu
from jax.experimental.pallas import tpu_sc as plsc

BATCH = 4096
MAXLEN = 200
EMBED = 32

NC = 2
NS = 16
NW = NC * NS

ROWS_PER_W = BATCH // NW
CHUNK_ROWS = 4
CHUNK = CHUNK_ROWS * MAXLEN
NCHUNK = ROWS_PER_W // CHUNK_ROWS
GATHER_W = 100
NGATHER = CHUNK // GATHER_W


def _embed_kernel(x_hbm, tok_hbm, pos_hbm, out_hbm,
                  idx_v, rows_v, pos_v, gsem0, gsem1, osem0, osem1):
    wid = lax.axis_index("c") * NS + lax.axis_index("s")
    gsems = (gsem0, gsem1)
    osems = (osem0, osem1)

    pltpu.sync_copy(pos_hbm, pos_v)

    def start_gathers(b):
        for j in range(NGATHER):
            r, half = divmod(j, 2)
            pltpu.async_copy(
                tok_hbm.at[idx_v.at[b, j]],
                rows_v.at[b, r, pl.ds(half * GATHER_W, GATHER_W)],
                gsems[b],
            )

    def wait_gathers(b):
        for j in range(NGATHER):
            r, half = divmod(j, 2)
            pltpu.make_async_copy(
                tok_hbm.at[idx_v.at[b, j]],
                rows_v.at[b, r, pl.ds(half * GATHER_W, GATHER_W)],
                gsems[b],
            ).wait()

    def wait_writeout(c, b):
        pltpu.make_async_copy(
            rows_v.at[b], out_hbm.at[pl.ds(c * CHUNK_ROWS, CHUNK_ROWS)],
            osems[b],
        ).wait()

    def add_pos(b):
        def add_body(t, carry2):
            p0 = pos_v[t, pl.ds(0, 16)]
            p1 = pos_v[t, pl.ds(16, 16)]
            for r in range(CHUNK_ROWS):
                rows_v[b, r, t, pl.ds(0, 16)] = (
                    rows_v[b, r, t, pl.ds(0, 16)] + p0)
                rows_v[b, r, t, pl.ds(16, 16)] = (
                    rows_v[b, r, t, pl.ds(16, 16)] + p1)
            return carry2

        lax.fori_loop(0, MAXLEN, add_body, 0)

    c0 = wid * NCHUNK
    pltpu.sync_copy(x_hbm.at[c0], idx_v.at[0])
    start_gathers(0)

    def pair_body(h, carry):
        for b in range(2):
            g = 2 * h + b
            c = c0 + g
            q = 1 - b
            wait_gathers(b)

            @pl.when(g < NCHUNK - 1)
            def _():
                pltpu.sync_copy(x_hbm.at[c + 1], idx_v.at[q])

                @pl.when(g >= 1)
                def _():
                    wait_writeout(c - 1, q)

                start_gathers(q)

            add_pos(b)
            pltpu.async_copy(
                rows_v.at[b],
                out_hbm.at[pl.ds(c * CHUNK_ROWS, CHUNK_ROWS)],
                osems[b],
            )
        return carry

    lax.fori_loop(0, NCHUNK // 2, pair_body, 0)

    wait_writeout(c0 + NCHUNK - 1, 1)


def kernel(x, token_table, pos_table):
    x3 = x.astype(jnp.int32).reshape(NW * NCHUNK, NGATHER, GATHER_W)
    mesh = plsc.VectorSubcoreMesh(core_axis_name="c", subcore_axis_name="s")
    run = functools.partial(
        pl.kernel,
        mesh=mesh,
        compiler_params=pltpu.CompilerParams(use_tc_tiling_on_sc=False),
        out_type=jax.ShapeDtypeStruct((BATCH, MAXLEN, EMBED), jnp.float32),
        scratch_types=[
            pltpu.VMEM((2, NGATHER, GATHER_W), jnp.int32),
            pltpu.VMEM((2, CHUNK_ROWS, MAXLEN, EMBED), jnp.float32),
            pltpu.VMEM((MAXLEN, EMBED), jnp.float32),
            pltpu.SemaphoreType.DMA,
            pltpu.SemaphoreType.DMA,
            pltpu.SemaphoreType.DMA,
            pltpu.SemaphoreType.DMA,
        ],
    )(_embed_kernel)
    return run(x3, token_table, pos_table)
